# baseline (speedup 1.0000x reference)
import functools

import jax
import jax.numpy as jnp
from jax import lax
from jax.experimental import pallas as pl
from jax.experimental.pallas import tpu as pltpu
from jax.experimental.pallas import tpu_sc as plsc

_NC = 2
_NS = 16
_LANES = 128


@functools.partial(jax.jit, static_argnames=("n", "acc_rows", "ch0", "ch1"))
def _sc_edge_agg(h, src2d, dst2d, *, n, acc_rows, ch0, ch1):
    hid = h.shape[1]
    acc_rows_per_tile = acc_rows // _NS
    mesh = plsc.VectorSubcoreMesh(
        core_axis_name="c", subcore_axis_name="s",
        num_cores=_NC, num_subcores=_NS)

    seg = 16
    assert ch0 % seg == 0 and ch1 % seg == 0

    @functools.partial(
        pl.kernel,
        out_type=jax.ShapeDtypeStruct((_NC * acc_rows, hid), jnp.float32),
        mesh=mesh,
        scratch_types=[
            pltpu.VMEM((seg, _LANES), jnp.int32),
            pltpu.VMEM((seg, _LANES), jnp.int32),
            pltpu.VMEM((seg, _LANES), jnp.int32),
            pltpu.VMEM((seg, _LANES), jnp.int32),
            pltpu.VMEM((_LANES, hid), jnp.float32),
            pltpu.VMEM((_LANES, hid), jnp.float32),
            pltpu.VMEM_SHARED((acc_rows, hid), jnp.float32),
            pltpu.SemaphoreType.DMA,
            pltpu.SemaphoreType.DMA,
            pltpu.SemaphoreType.DMA,
        ],
    )
    def agg(h_hbm, src_hbm, dst_hbm, out_hbm,
            src0, src1, dst0, dst1, buf0, buf1, acc_sh, g0, g1, isem):
        srcb = (src0, src1)
        dstb = (dst0, dst1)
        bufs = (buf0, buf1)
        gsems = (g0, g1)
        c = lax.axis_index("c")
        s = lax.axis_index("s")

        zv = jnp.zeros((16,), jnp.float32)

        def zrow(r, carry):
            for k in range(hid // 16):
                buf0[r, pl.ds(k * 16, 16)] = zv
            return carry

        lax.fori_loop(0, _LANES, zrow, 0)
        zbase = s * acc_rows_per_tile
        for k in range(acc_rows_per_tile // _LANES):
            pltpu.sync_copy(buf0,
                            acc_sh.at[pl.ds(zbase + k * _LANES, _LANES)])
        rem = acc_rows_per_tile % _LANES
        if rem:
            full = acc_rows_per_tile - rem
            pltpu.sync_copy(buf0.at[pl.ds(0, rem)],
                            acc_sh.at[pl.ds(zbase + full, rem)])
        plsc.subcore_barrier()

        def step(src_v, dst_v, j, b, prefetch):
            pltpu.make_async_copy(h_hbm.at[src_v.at[j]], bufs[b],
                                  gsems[b]).wait()
            pltpu.sync_copy(bufs[b], acc_sh.at[dst_v.at[j]], add=True)
            if prefetch:
                pltpu.async_copy(h_hbm.at[src_v.at[j + 2]], bufs[b],
                                 gsems[b])

        def run(nseg, ebase):
            pltpu.sync_copy(src_hbm.at[pl.ds(ebase, seg)], src0)
            pltpu.sync_copy(dst_hbm.at[pl.ds(ebase, seg)], dst0)
            for sg in range(nseg):
                p = sg % 2
                src_v, dst_v = srcb[p], dstb[p]
                if sg > 0:
                    pltpu.make_async_copy(
                        src_hbm.at[pl.ds(ebase, seg)], src_v, isem).wait()
                    pltpu.make_async_copy(
                        dst_hbm.at[pl.ds(ebase, seg)], dst_v, isem).wait()
                if sg + 1 < nseg:
                    off = ebase + (sg + 1) * seg
                    pltpu.async_copy(src_hbm.at[pl.ds(off, seg)],
                                     srcb[1 - p], isem)
                    pltpu.async_copy(dst_hbm.at[pl.ds(off, seg)],
                                     dstb[1 - p], isem)
                pltpu.async_copy(h_hbm.at[src_v.at[0]], bufs[0], gsems[0])
                pltpu.async_copy(h_hbm.at[src_v.at[1]], bufs[1], gsems[1])

                def body(i, carry):
                    step(src_v, dst_v, 2 * i, 0, prefetch=True)
                    step(src_v, dst_v, 2 * i + 1, 1, prefetch=True)
                    return carry

                lax.fori_loop(0, seg // 2 - 1, body, 0)
                step(src_v, dst_v, seg - 2, 0, prefetch=False)
                step(src_v, dst_v, seg - 1, 1, prefetch=False)

        pl.when(c == 0)(lambda: run(ch0 // seg, s * ch0))
        pl.when(c == 1)(lambda: run(ch1 // seg, _NS * ch0 + s * ch1))
        plsc.subcore_barrier()

        base = c * acc_rows + s * acc_rows_per_tile
        pltpu.sync_copy(acc_sh.at[pl.ds(s * acc_rows_per_tile, acc_rows_per_tile)],
                        out_hbm.at[pl.ds(base, acc_rows_per_tile)])

    out = agg(h, src2d, dst2d)
    return out.reshape(_NC, acc_rows, hid)[:, :n, :]


def _emb_body(h_ref, w_ref, b_ref, o_ref):
    o_ref[...] = jnp.dot(h_ref[...], w_ref[...],
                         preferred_element_type=jnp.float32) + b_ref[...]


def _tc_emb(h, w, b):
    return pl.pallas_call(
        _emb_body,
        out_shape=jax.ShapeDtypeStruct((h.shape[0], w.shape[1]), jnp.float32),
    )(h, w, b.reshape(1, -1))


def _bn(x, gb_ref):
    g = gb_ref[0:1, :]
    b = gb_ref[1:2, :]
    m = jnp.mean(x, axis=0, keepdims=True)
    d = x - m
    v = jnp.mean(d * d, axis=0, keepdims=True)
    return g * (d * lax.rsqrt(v + 1e-5)) + b


def _layer_body(h_ref, p0_ref, p1_ref, eps_ref, w1_ref, b1_ref,
                w2_ref, b2_ref, bn1_ref, bn2_ref, bn3_ref, o_ref):
    h = h_ref[...]
    hh = eps_ref[...] * h + (p0_ref[...] + p1_ref[...])
    y = jnp.dot(hh, w1_ref[...], preferred_element_type=jnp.float32) + b1_ref[...]
    y = jnp.maximum(_bn(y, bn1_ref), 0.0)
    y = jnp.dot(y, w2_ref[...], preferred_element_type=jnp.float32) + b2_ref[...]
    y = jnp.maximum(_bn(y, bn2_ref), 0.0)
    y = jnp.maximum(_bn(y, bn3_ref), 0.0)
    o_ref[...] = h + y


def _tc_layer(h, p0, p1, lp):
    (w1, b1), (w2, b2) = lp['mlp']
    epsp = (1.0 + lp['eps']).reshape(1, 1)
    bn1 = jnp.stack(lp['mlp_bn'])
    bn2 = jnp.stack(lp['apply_bn'])
    bn3 = jnp.stack(lp['layer_bn'])
    return pl.pallas_call(
        _layer_body,
        out_shape=jax.ShapeDtypeStruct(h.shape, jnp.float32),
    )(h, p0, p1, epsp, w1, b1.reshape(1, -1), w2, b2.reshape(1, -1),
      bn1, bn2, bn3)


def _readout_body(ids_ref, h0_ref, h1_ref, h2_ref, h3_ref, h4_ref,
                  wp_ref, bp_ref, wa_ref, ba_ref, wc_ref, bc_ref,
                  wo_ref, bo_ref, xc_ref, xo_ref, xco_ref):
    g_count = xc_ref.shape[0]
    n = ids_ref.shape[1]
    gi = lax.broadcasted_iota(jnp.int32, (g_count, n), 0)
    p_t = (gi == ids_ref[...]).astype(jnp.float32)

    score = jnp.zeros_like(xo_ref)
    hs = (h0_ref, h1_ref, h2_ref, h3_ref, h4_ref)
    for i in range(5):
        pooled = jnp.dot(p_t, hs[i][...], preferred_element_type=jnp.float32)
        score = score + jnp.dot(pooled, wp_ref[i],
                                preferred_element_type=jnp.float32) + bp_ref[i]

    hlast = h4_ref[...]
    logits = jnp.dot(hlast, wa_ref[...],
                     preferred_element_type=jnp.float32) + ba_ref[...]
    m = jnp.max(logits, axis=1, keepdims=True)
    e = jnp.exp(logits - m)
    att = e / jnp.sum(e, axis=1, keepdims=True)
    hc = jnp.dot(p_t, att[:, 0:1] * hlast, preferred_element_type=jnp.float32)
    ho = jnp.dot(p_t, att[:, 1:2] * hlast, preferred_element_type=jnp.float32)

    xc_ref[...] = jnp.dot(hc, wc_ref[...],
                          preferred_element_type=jnp.float32) + bc_ref[...]
    xo = jnp.dot(ho, wo_ref[...],
                 preferred_element_type=jnp.float32) + bo_ref[...] + score
    xo_ref[...] = xo
    hcr = jnp.concatenate([hc[g_count - 1:g_count], hc[:g_count - 1]], axis=0)
    xco_ref[...] = jnp.dot(ho + hcr, wo_ref[...],
                           preferred_element_type=jnp.float32) \
        + bo_ref[...] + score


def _tc_readout(ids_row, hidden, params, g_count, ncls):
    wp = jnp.stack([w for (w, _) in params['pred']])
    bp = jnp.stack([b for (_, b) in params['pred']])[:, None, :]
    wa, ba = params['att']
    wc, bc = params['lc']
    wo, bo = params['lo']
    out_sh = jax.ShapeDtypeStruct((g_count, ncls), jnp.float32)
    return pl.pallas_call(
        _readout_body,
        out_shape=(out_sh, out_sh, out_sh),
    )(ids_row, *hidden, wp, bp, wa, ba.reshape(1, -1), wc, bc.reshape(1, -1),
      wo, bo.reshape(1, -1))


def kernel(h, e, params, edge_index, node_graph_ids):
    n, _ = h.shape
    n_edges = edge_index.shape[1]
    g_count = 128
    ncls = params['lc'][0].shape[1]

    per_op = _NC * _NS * _LANES
    ch = -(-n_edges // (per_op * 8)) * 8
    e_pad = ch * per_op
    ch1 = (2 * ch) * 3 // 10 // 16 * 16
    ch0 = 2 * ch - ch1
    src = edge_index[0].astype(jnp.int32)
    dst = edge_index[1].astype(jnp.int32)
    pad = e_pad - n_edges
    src2d = jnp.concatenate([src, jnp.zeros((pad,), jnp.int32)]).reshape(-1, _LANES)
    dst2d = jnp.concatenate([dst, jnp.full((pad,), n, jnp.int32)]).reshape(-1, _LANES)

    acc_rows = -(-(n + 1) // (_NS * 8)) * (_NS * 8)

    w_emb, b_emb = params['emb']
    hcur = _tc_emb(h, w_emb, b_emb)
    hidden = [hcur]
    for lp in params['gin']:
        partials = _sc_edge_agg(hcur, src2d, dst2d,
                                n=n, acc_rows=acc_rows, ch0=ch0, ch1=ch1)
        hcur = _tc_layer(hcur, partials[0], partials[1], lp)
        hidden.append(hcur)

    ids_row = node_graph_ids.astype(jnp.int32).reshape(1, n)
    return _tc_readout(ids_row, hidden, params, g_count, ncls)


# speedup vs baseline: 2.9753x; 2.9753x over previous
import functools

import jax
import jax.numpy as jnp
from jax import lax
from jax.experimental import pallas as pl
from jax.experimental.pallas import tpu as pltpu
from jax.experimental.pallas import tpu_sc as plsc

_NC = 2
_NS = 16
_LANES = 128


@functools.partial(jax.jit, static_argnames=("n", "acc_rows", "ch0", "ch1"))
def _sc_edge_agg(h, src2d, dst2d, *, n, acc_rows, ch0, ch1):
    hid = h.shape[1]
    acc_rows_per_tile = acc_rows // _NS
    mesh = plsc.VectorSubcoreMesh(
        core_axis_name="c", subcore_axis_name="s",
        num_cores=_NC, num_subcores=_NS)

    seg = 16
    assert ch0 % seg == 0 and ch1 % seg == 0

    @functools.partial(
        pl.kernel,
        out_type=jax.ShapeDtypeStruct((_NC * acc_rows, hid), jnp.float32),
        mesh=mesh,
        scratch_types=[
            pltpu.VMEM((seg, _LANES), jnp.int32),
            pltpu.VMEM((seg, _LANES), jnp.int32),
            pltpu.VMEM((seg, _LANES), jnp.int32),
            pltpu.VMEM((seg, _LANES), jnp.int32),
            pltpu.VMEM((_LANES, hid), jnp.float32),
            pltpu.VMEM((_LANES, hid), jnp.float32),
            pltpu.VMEM_SHARED((acc_rows, hid), jnp.float32),
            pltpu.SemaphoreType.DMA,
            pltpu.SemaphoreType.DMA,
            pltpu.SemaphoreType.DMA,
        ],
    )
    def agg(h_hbm, src_hbm, dst_hbm, out_hbm,
            src0, src1, dst0, dst1, buf0, buf1, acc_sh, g0, g1, isem):
        srcb = (src0, src1)
        dstb = (dst0, dst1)
        bufs = (buf0, buf1)
        gsems = (g0, g1)
        c = lax.axis_index("c")
        s = lax.axis_index("s")

        zv = jnp.zeros((16,), jnp.float32)

        def zrow(r, carry):
            for k in range(hid // 16):
                buf0[r, pl.ds(k * 16, 16)] = zv
            return carry

        lax.fori_loop(0, _LANES, zrow, 0)
        zbase = s * acc_rows_per_tile
        for k in range(acc_rows_per_tile // _LANES):
            pltpu.sync_copy(buf0,
                            acc_sh.at[pl.ds(zbase + k * _LANES, _LANES)])
        rem = acc_rows_per_tile % _LANES
        if rem:
            full = acc_rows_per_tile - rem
            pltpu.sync_copy(buf0.at[pl.ds(0, rem)],
                            acc_sh.at[pl.ds(zbase + full, rem)])
        plsc.subcore_barrier()

        def step(src_v, dst_v, j, b, prefetch):
            pltpu.make_async_copy(h_hbm.at[src_v.at[j]], bufs[b],
                                  gsems[b]).wait()
            pltpu.sync_copy(bufs[b], acc_sh.at[dst_v.at[j]], add=True)
            if prefetch:
                pltpu.async_copy(h_hbm.at[src_v.at[j + 2]], bufs[b],
                                 gsems[b])

        def run(nseg, ebase):
            pltpu.sync_copy(src_hbm.at[pl.ds(ebase, seg)], src0)
            pltpu.sync_copy(dst_hbm.at[pl.ds(ebase, seg)], dst0)
            for sg in range(nseg):
                p = sg % 2
                src_v, dst_v = srcb[p], dstb[p]
                if sg > 0:
                    pltpu.make_async_copy(
                        src_hbm.at[pl.ds(ebase, seg)], src_v, isem).wait()
                    pltpu.make_async_copy(
                        dst_hbm.at[pl.ds(ebase, seg)], dst_v, isem).wait()
                if sg + 1 < nseg:
                    off = ebase + (sg + 1) * seg
                    pltpu.async_copy(src_hbm.at[pl.ds(off, seg)],
                                     srcb[1 - p], isem)
                    pltpu.async_copy(dst_hbm.at[pl.ds(off, seg)],
                                     dstb[1 - p], isem)
                pltpu.async_copy(h_hbm.at[src_v.at[0]], bufs[0], gsems[0])
                pltpu.async_copy(h_hbm.at[src_v.at[1]], bufs[1], gsems[1])

                def body(i, carry):
                    step(src_v, dst_v, 2 * i, 0, prefetch=True)
                    step(src_v, dst_v, 2 * i + 1, 1, prefetch=True)
                    return carry

                lax.fori_loop(0, seg // 2 - 1, body, 0)
                step(src_v, dst_v, seg - 2, 0, prefetch=False)
                step(src_v, dst_v, seg - 1, 1, prefetch=False)

        pl.when(c == 0)(lambda: run(ch0 // seg, s * ch0))
        pl.when(c == 1)(lambda: run(ch1 // seg, _NS * ch0 + s * ch1))
        plsc.subcore_barrier()

        base = c * acc_rows + s * acc_rows_per_tile
        pltpu.sync_copy(acc_sh.at[pl.ds(s * acc_rows_per_tile, acc_rows_per_tile)],
                        out_hbm.at[pl.ds(base, acc_rows_per_tile)])

    out = agg(h, src2d, dst2d)
    return out.reshape(_NC, acc_rows, hid)[:, :n, :]


def _emb_body(h_ref, w_ref, b_ref, o_ref):
    o_ref[...] = jnp.dot(h_ref[...], w_ref[...],
                         preferred_element_type=jnp.float32) + b_ref[...]


def _tc_emb(h, w, b):
    return pl.pallas_call(
        _emb_body,
        out_shape=jax.ShapeDtypeStruct((h.shape[0], w.shape[1]), jnp.float32),
    )(h, w, b.reshape(1, -1))


def _bn(x, gb_ref):
    g = gb_ref[0:1, :]
    b = gb_ref[1:2, :]
    m = jnp.mean(x, axis=0, keepdims=True)
    d = x - m
    v = jnp.mean(d * d, axis=0, keepdims=True)
    return g * (d * lax.rsqrt(v + 1e-5)) + b


def _layer_body(h_ref, p0_ref, p1_ref, eps_ref, w1_ref, b1_ref,
                w2_ref, b2_ref, bn1_ref, bn2_ref, bn3_ref, o_ref):
    h = h_ref[...]
    hh = eps_ref[...] * h + (p0_ref[...] + p1_ref[...])
    y = jnp.dot(hh, w1_ref[...], preferred_element_type=jnp.float32) + b1_ref[...]
    y = jnp.maximum(_bn(y, bn1_ref), 0.0)
    y = jnp.dot(y, w2_ref[...], preferred_element_type=jnp.float32) + b2_ref[...]
    y = jnp.maximum(_bn(y, bn2_ref), 0.0)
    y = jnp.maximum(_bn(y, bn3_ref), 0.0)
    o_ref[...] = h + y


def _tc_layer(h, p0, p1, lp):
    (w1, b1), (w2, b2) = lp['mlp']
    epsp = (1.0 + lp['eps']).reshape(1, 1)
    bn1 = jnp.stack(lp['mlp_bn'])
    bn2 = jnp.stack(lp['apply_bn'])
    bn3 = jnp.stack(lp['layer_bn'])
    return pl.pallas_call(
        _layer_body,
        out_shape=jax.ShapeDtypeStruct(h.shape, jnp.float32),
    )(h, p0, p1, epsp, w1, b1.reshape(1, -1), w2, b2.reshape(1, -1),
      bn1, bn2, bn3)


def _readout_body(ids_ref, h0_ref, h1_ref, h2_ref, h3_ref, h4_ref,
                  wp_ref, bp_ref, wa_ref, ba_ref, wc_ref, bc_ref,
                  wo_ref, bo_ref, xc_ref, xo_ref, xco_ref):
    g_count = xc_ref.shape[0]
    n = ids_ref.shape[1]
    gi = lax.broadcasted_iota(jnp.int32, (g_count, n), 0)
    p_t = (gi == ids_ref[...]).astype(jnp.float32)

    score = jnp.zeros_like(xo_ref)
    hs = (h0_ref, h1_ref, h2_ref, h3_ref, h4_ref)
    for i in range(5):
        pooled = jnp.dot(p_t, hs[i][...], preferred_element_type=jnp.float32)
        score = score + jnp.dot(pooled, wp_ref[i],
                                preferred_element_type=jnp.float32) + bp_ref[i]

    hlast = h4_ref[...]
    logits = jnp.dot(hlast, wa_ref[...],
                     preferred_element_type=jnp.float32) + ba_ref[...]
    m = jnp.max(logits, axis=1, keepdims=True)
    e = jnp.exp(logits - m)
    att = e / jnp.sum(e, axis=1, keepdims=True)
    hc = jnp.dot(p_t, att[:, 0:1] * hlast, preferred_element_type=jnp.float32)
    ho = jnp.dot(p_t, att[:, 1:2] * hlast, preferred_element_type=jnp.float32)

    xc_ref[...] = jnp.dot(hc, wc_ref[...],
                          preferred_element_type=jnp.float32) + bc_ref[...]
    xo = jnp.dot(ho, wo_ref[...],
                 preferred_element_type=jnp.float32) + bo_ref[...] + score
    xo_ref[...] = xo
    hcr = jnp.concatenate([hc[g_count - 1:g_count], hc[:g_count - 1]], axis=0)
    xco_ref[...] = jnp.dot(ho + hcr, wo_ref[...],
                           preferred_element_type=jnp.float32) \
        + bo_ref[...] + score


def _tc_readout(ids_row, hidden, params, g_count, ncls):
    wp = jnp.stack([w for (w, _) in params['pred']])
    bp = jnp.stack([b for (_, b) in params['pred']])[:, None, :]
    wa, ba = params['att']
    wc, bc = params['lc']
    wo, bo = params['lo']
    out_sh = jax.ShapeDtypeStruct((g_count, ncls), jnp.float32)
    return pl.pallas_call(
        _readout_body,
        out_shape=(out_sh, out_sh, out_sh),
    )(ids_row, *hidden, wp, bp, wa, ba.reshape(1, -1), wc, bc.reshape(1, -1),
      wo, bo.reshape(1, -1))


def kernel(h, e, params, edge_index, node_graph_ids):
    n, _ = h.shape
    n_edges = edge_index.shape[1]
    g_count = 128
    ncls = params['lc'][0].shape[1]

    per_op = _NC * _NS * _LANES
    ch = -(-n_edges // (per_op * 8)) * 8
    e_pad = ch * per_op
    ch1 = ch
    ch0 = 2 * ch - ch1
    src = edge_index[0].astype(jnp.int32)
    dst = edge_index[1].astype(jnp.int32)
    pad = e_pad - n_edges
    acc_rows = -(-(n + 1) // (_NS * 8)) * (_NS * 8)
    pad_src = jnp.arange(pad, dtype=jnp.int32) % jnp.int32(n)
    pad_dst = n + jnp.arange(pad, dtype=jnp.int32) % jnp.int32(acc_rows - n)
    src2d = jnp.concatenate([src, pad_src]).reshape(-1, _LANES)
    dst2d = jnp.concatenate([dst, pad_dst]).reshape(-1, _LANES)

    w_emb, b_emb = params['emb']
    hcur = _tc_emb(h, w_emb, b_emb)
    hidden = [hcur]
    for lp in params['gin']:
        partials = _sc_edge_agg(hcur, src2d, dst2d,
                                n=n, acc_rows=acc_rows, ch0=ch0, ch1=ch1)
        hcur = _tc_layer(hcur, partials[0], partials[1], lp)
        hidden.append(hcur)

    ids_row = node_graph_ids.astype(jnp.int32).reshape(1, n)
    return _tc_readout(ids_row, hidden, params, g_count, ncls)


# speedup vs baseline: 3.1137x; 1.0465x over previous
import functools

import jax
import jax.numpy as jnp
from jax import lax
from jax.experimental import pallas as pl
from jax.experimental.pallas import tpu as pltpu
from jax.experimental.pallas import tpu_sc as plsc

_NC = 2
_NS = 16
_LANES = 128


@functools.partial(jax.jit, static_argnames=("n", "acc_rows", "ch0", "ch1"))
def _sc_edge_agg(h, src2d, dst2d, *, n, acc_rows, ch0, ch1):
    hid = h.shape[1]
    acc_rows_per_tile = acc_rows // _NS
    mesh = plsc.VectorSubcoreMesh(
        core_axis_name="c", subcore_axis_name="s",
        num_cores=_NC, num_subcores=_NS)

    seg = 16
    assert ch0 % seg == 0 and ch1 % seg == 0

    @functools.partial(
        pl.kernel,
        out_type=jax.ShapeDtypeStruct((_NC * acc_rows, hid), jnp.float32),
        mesh=mesh,
        scratch_types=[
            pltpu.VMEM((seg, _LANES), jnp.int32),
            pltpu.VMEM((seg, _LANES), jnp.int32),
            pltpu.VMEM((seg, _LANES), jnp.int32),
            pltpu.VMEM((seg, _LANES), jnp.int32),
            pltpu.VMEM((_LANES, hid), jnp.float32),
            pltpu.VMEM((_LANES, hid), jnp.float32),
            pltpu.VMEM_SHARED((acc_rows, hid), jnp.float32),
            pltpu.SemaphoreType.DMA,
            pltpu.SemaphoreType.DMA,
            pltpu.SemaphoreType.DMA,
        ],
    )
    def agg(h_hbm, src_hbm, dst_hbm, out_hbm,
            src0, src1, dst0, dst1, buf0, buf1, acc_sh, g0, g1, isem):
        srcb = (src0, src1)
        dstb = (dst0, dst1)
        bufs = (buf0, buf1)
        gsems = (g0, g1)
        c = lax.axis_index("c")
        s = lax.axis_index("s")

        zv = jnp.zeros((16,), jnp.float32)

        def zrow(r, carry):
            for k in range(hid // 16):
                buf0[r, pl.ds(k * 16, 16)] = zv
            return carry

        lax.fori_loop(0, _LANES, zrow, 0)
        zbase = s * acc_rows_per_tile
        for k in range(acc_rows_per_tile // _LANES):
            pltpu.sync_copy(buf0,
                            acc_sh.at[pl.ds(zbase + k * _LANES, _LANES)])
        rem = acc_rows_per_tile % _LANES
        if rem:
            full = acc_rows_per_tile - rem
            pltpu.sync_copy(buf0.at[pl.ds(0, rem)],
                            acc_sh.at[pl.ds(zbase + full, rem)])
        plsc.subcore_barrier()

        def step(src_v, dst_v, j, b, prefetch):
            pltpu.make_async_copy(h_hbm.at[src_v.at[j]], bufs[b],
                                  gsems[b]).wait()
            pltpu.sync_copy(bufs[b], acc_sh.at[dst_v.at[j]], add=True)
            if prefetch:
                pltpu.async_copy(h_hbm.at[src_v.at[j + 2]], bufs[b],
                                 gsems[b])

        def run(nseg, ebase):
            pltpu.sync_copy(src_hbm.at[pl.ds(ebase, seg)], src0)
            pltpu.sync_copy(dst_hbm.at[pl.ds(ebase, seg)], dst0)
            for sg in range(nseg):
                p = sg % 2
                src_v, dst_v = srcb[p], dstb[p]
                if sg > 0:
                    pltpu.make_async_copy(
                        src_hbm.at[pl.ds(ebase, seg)], src_v, isem).wait()
                    pltpu.make_async_copy(
                        dst_hbm.at[pl.ds(ebase, seg)], dst_v, isem).wait()
                if sg + 1 < nseg:
                    off = ebase + (sg + 1) * seg
                    pltpu.async_copy(src_hbm.at[pl.ds(off, seg)],
                                     srcb[1 - p], isem)
                    pltpu.async_copy(dst_hbm.at[pl.ds(off, seg)],
                                     dstb[1 - p], isem)
                pltpu.async_copy(h_hbm.at[src_v.at[0]], bufs[0], gsems[0])
                pltpu.async_copy(h_hbm.at[src_v.at[1]], bufs[1], gsems[1])

                def body(i, carry):
                    step(src_v, dst_v, 2 * i, 0, prefetch=True)
                    step(src_v, dst_v, 2 * i + 1, 1, prefetch=True)
                    return carry

                lax.fori_loop(0, seg // 2 - 1, body, 0)
                step(src_v, dst_v, seg - 2, 0, prefetch=False)
                step(src_v, dst_v, seg - 1, 1, prefetch=False)

        pl.when(c == 0)(lambda: run(ch0 // seg, s * ch0))
        pl.when(c == 1)(lambda: run(ch1 // seg, _NS * ch0 + s * ch1))
        plsc.subcore_barrier()

        base = c * acc_rows + s * acc_rows_per_tile
        pltpu.sync_copy(acc_sh.at[pl.ds(s * acc_rows_per_tile, acc_rows_per_tile)],
                        out_hbm.at[pl.ds(base, acc_rows_per_tile)])

    return agg(h, src2d, dst2d)


def _emb_body(h_ref, w_ref, b_ref, o_ref):
    o_ref[...] = jnp.dot(h_ref[...], w_ref[...],
                         preferred_element_type=jnp.float32) + b_ref[...]


def _tc_emb(h, w, b):
    return pl.pallas_call(
        _emb_body,
        out_shape=jax.ShapeDtypeStruct((h.shape[0], w.shape[1]), jnp.float32),
    )(h, w, b.reshape(1, -1))


def _bn(x, gb_ref):
    g = gb_ref[0:1, :]
    b = gb_ref[1:2, :]
    m = jnp.mean(x, axis=0, keepdims=True)
    d = x - m
    v = jnp.mean(d * d, axis=0, keepdims=True)
    return g * (d * lax.rsqrt(v + 1e-5)) + b


def _layer_body(h_ref, p_ref, eps_ref, w1_ref, b1_ref,
                w2_ref, b2_ref, bn1_ref, bn2_ref, bn3_ref, o_ref):
    h = h_ref[...]
    n_rows = h_ref.shape[0]
    acc_rows = p_ref.shape[0] // 2
    p0 = p_ref[pl.ds(0, n_rows), :]
    p1 = p_ref[pl.ds(acc_rows, n_rows), :]
    hh = eps_ref[...] * h + (p0 + p1)
    y = jnp.dot(hh, w1_ref[...], preferred_element_type=jnp.float32) + b1_ref[...]
    y = jnp.maximum(_bn(y, bn1_ref), 0.0)
    y = jnp.dot(y, w2_ref[...], preferred_element_type=jnp.float32) + b2_ref[...]
    y = jnp.maximum(_bn(y, bn2_ref), 0.0)
    y = jnp.maximum(_bn(y, bn3_ref), 0.0)
    o_ref[...] = h + y


def _tc_layer(h, pfull, lp):
    (w1, b1), (w2, b2) = lp['mlp']
    epsp = (1.0 + lp['eps']).reshape(1, 1)
    bn1 = jnp.stack(lp['mlp_bn'])
    bn2 = jnp.stack(lp['apply_bn'])
    bn3 = jnp.stack(lp['layer_bn'])
    return pl.pallas_call(
        _layer_body,
        out_shape=jax.ShapeDtypeStruct(h.shape, jnp.float32),
    )(h, pfull, epsp, w1, b1.reshape(1, -1), w2, b2.reshape(1, -1),
      bn1, bn2, bn3)


def _readout_body(ids_ref, h0_ref, h1_ref, h2_ref, h3_ref, h4_ref,
                  wp_ref, bp_ref, wa_ref, ba_ref, wc_ref, bc_ref,
                  wo_ref, bo_ref, xc_ref, xo_ref, xco_ref):
    g_count = xc_ref.shape[0]
    n = ids_ref.shape[1]
    gi = lax.broadcasted_iota(jnp.int32, (g_count, n), 0)
    p_t = (gi == ids_ref[...]).astype(jnp.float32)

    score = jnp.zeros_like(xo_ref)
    hs = (h0_ref, h1_ref, h2_ref, h3_ref, h4_ref)
    for i in range(5):
        pooled = jnp.dot(p_t, hs[i][...], preferred_element_type=jnp.float32)
        score = score + jnp.dot(pooled, wp_ref[i],
                                preferred_element_type=jnp.float32) + bp_ref[i]

    hlast = h4_ref[...]
    logits = jnp.dot(hlast, wa_ref[...],
                     preferred_element_type=jnp.float32) + ba_ref[...]
    m = jnp.max(logits, axis=1, keepdims=True)
    e = jnp.exp(logits - m)
    att = e / jnp.sum(e, axis=1, keepdims=True)
    hc = jnp.dot(p_t, att[:, 0:1] * hlast, preferred_element_type=jnp.float32)
    ho = jnp.dot(p_t, att[:, 1:2] * hlast, preferred_element_type=jnp.float32)

    xc_ref[...] = jnp.dot(hc, wc_ref[...],
                          preferred_element_type=jnp.float32) + bc_ref[...]
    xo = jnp.dot(ho, wo_ref[...],
                 preferred_element_type=jnp.float32) + bo_ref[...] + score
    xo_ref[...] = xo
    hcr = jnp.concatenate([hc[g_count - 1:g_count], hc[:g_count - 1]], axis=0)
    xco_ref[...] = jnp.dot(ho + hcr, wo_ref[...],
                           preferred_element_type=jnp.float32) \
        + bo_ref[...] + score


def _tc_readout(ids_row, hidden, params, g_count, ncls):
    wp = jnp.stack([w for (w, _) in params['pred']])
    bp = jnp.stack([b for (_, b) in params['pred']])[:, None, :]
    wa, ba = params['att']
    wc, bc = params['lc']
    wo, bo = params['lo']
    out_sh = jax.ShapeDtypeStruct((g_count, ncls), jnp.float32)
    return pl.pallas_call(
        _readout_body,
        out_shape=(out_sh, out_sh, out_sh),
    )(ids_row, *hidden, wp, bp, wa, ba.reshape(1, -1), wc, bc.reshape(1, -1),
      wo, bo.reshape(1, -1))


def kernel(h, e, params, edge_index, node_graph_ids):
    n, _ = h.shape
    n_edges = edge_index.shape[1]
    g_count = 128
    ncls = params['lc'][0].shape[1]

    per_op = _NC * _NS * _LANES
    ch = -(-n_edges // (per_op * 8)) * 8
    e_pad = ch * per_op
    ch1 = ch
    ch0 = 2 * ch - ch1
    src = edge_index[0].astype(jnp.int32)
    dst = edge_index[1].astype(jnp.int32)
    pad = e_pad - n_edges
    acc_rows = -(-(n + 1) // (_NS * 8)) * (_NS * 8)
    pad_src = jnp.arange(pad, dtype=jnp.int32) % jnp.int32(n)
    pad_dst = n + jnp.arange(pad, dtype=jnp.int32) % jnp.int32(acc_rows - n)
    src2d = jnp.concatenate([src, pad_src]).reshape(-1, _LANES)
    dst2d = jnp.concatenate([dst, pad_dst]).reshape(-1, _LANES)

    w_emb, b_emb = params['emb']
    hcur = _tc_emb(h, w_emb, b_emb)
    hidden = [hcur]
    for lp in params['gin']:
        partials = _sc_edge_agg(hcur, src2d, dst2d,
                                n=n, acc_rows=acc_rows, ch0=ch0, ch1=ch1)
        hcur = _tc_layer(hcur, partials, lp)
        hidden.append(hcur)

    ids_row = node_graph_ids.astype(jnp.int32).reshape(1, n)
    return _tc_readout(ids_row, hidden, params, g_count, ncls)


# speedup vs baseline: 3.2476x; 1.0430x over previous
import functools

import jax
import jax.numpy as jnp
from jax import lax
from jax.experimental import pallas as pl
from jax.experimental.pallas import tpu as pltpu
from jax.experimental.pallas import tpu_sc as plsc

_NC = 2
_NS = 16
_LANES = 128


@functools.partial(jax.jit, static_argnames=("n", "acc_rows", "ch0", "ch1"))
def _sc_edge_agg(h, src2d, dst2d, *, n, acc_rows, ch0, ch1):
    hid = h.shape[1]
    acc_rows_per_tile = acc_rows // _NS
    mesh = plsc.VectorSubcoreMesh(
        core_axis_name="c", subcore_axis_name="s",
        num_cores=_NC, num_subcores=_NS)

    seg = 16
    assert ch0 % seg == 0 and ch1 % seg == 0

    @functools.partial(
        pl.kernel,
        out_type=jax.ShapeDtypeStruct((_NC * acc_rows, hid), jnp.float32),
        mesh=mesh,
        scratch_types=[
            pltpu.VMEM((seg, _LANES), jnp.int32),
            pltpu.VMEM((seg, _LANES), jnp.int32),
            pltpu.VMEM((seg, _LANES), jnp.int32),
            pltpu.VMEM((seg, _LANES), jnp.int32),
            pltpu.VMEM((_LANES, hid), jnp.float32),
            pltpu.VMEM((_LANES, hid), jnp.float32),
            pltpu.VMEM_SHARED((acc_rows, hid), jnp.float32),
            pltpu.SemaphoreType.DMA,
            pltpu.SemaphoreType.DMA,
            pltpu.SemaphoreType.DMA,
        ],
    )
    def agg(h_hbm, src_hbm, dst_hbm, out_hbm,
            src0, src1, dst0, dst1, buf0, buf1, acc_sh, g0, g1, isem):
        srcb = (src0, src1)
        dstb = (dst0, dst1)
        bufs = (buf0, buf1)
        gsems = (g0, g1)
        c = lax.axis_index("c")
        s = lax.axis_index("s")

        zv = jnp.zeros((16,), jnp.float32)

        def zrow(r, carry):
            for k in range(hid // 16):
                buf0[r, pl.ds(k * 16, 16)] = zv
            return carry

        lax.fori_loop(0, _LANES, zrow, 0)
        zbase = s * acc_rows_per_tile
        for k in range(acc_rows_per_tile // _LANES):
            pltpu.sync_copy(buf0,
                            acc_sh.at[pl.ds(zbase + k * _LANES, _LANES)])
        rem = acc_rows_per_tile % _LANES
        if rem:
            full = acc_rows_per_tile - rem
            pltpu.sync_copy(buf0.at[pl.ds(0, rem)],
                            acc_sh.at[pl.ds(zbase + full, rem)])
        plsc.subcore_barrier()

        def step(src_v, dst_v, j, b, prefetch):
            pltpu.make_async_copy(h_hbm.at[src_v.at[j]], bufs[b],
                                  gsems[b]).wait()
            pltpu.sync_copy(bufs[b], acc_sh.at[dst_v.at[j]], add=True)
            if prefetch:
                pltpu.async_copy(h_hbm.at[src_v.at[j + 2]], bufs[b],
                                 gsems[b])

        def run(nseg, ebase):
            pltpu.sync_copy(src_hbm.at[pl.ds(ebase, seg)], src0)
            pltpu.sync_copy(dst_hbm.at[pl.ds(ebase, seg)], dst0)
            for sg in range(nseg):
                p = sg % 2
                src_v, dst_v = srcb[p], dstb[p]
                if sg > 0:
                    pltpu.make_async_copy(
                        src_hbm.at[pl.ds(ebase, seg)], src_v, isem).wait()
                    pltpu.make_async_copy(
                        dst_hbm.at[pl.ds(ebase, seg)], dst_v, isem).wait()
                if sg + 1 < nseg:
                    off = ebase + (sg + 1) * seg
                    pltpu.async_copy(src_hbm.at[pl.ds(off, seg)],
                                     srcb[1 - p], isem)
                    pltpu.async_copy(dst_hbm.at[pl.ds(off, seg)],
                                     dstb[1 - p], isem)
                pltpu.async_copy(h_hbm.at[src_v.at[0]], bufs[0], gsems[0])
                pltpu.async_copy(h_hbm.at[src_v.at[1]], bufs[1], gsems[1])

                def body(i, carry):
                    step(src_v, dst_v, 2 * i, 0, prefetch=True)
                    step(src_v, dst_v, 2 * i + 1, 1, prefetch=True)
                    return carry

                lax.fori_loop(0, seg // 2 - 1, body, 0)
                step(src_v, dst_v, seg - 2, 0, prefetch=False)
                step(src_v, dst_v, seg - 1, 1, prefetch=False)

        pl.when(c == 0)(lambda: run(ch0 // seg, s * ch0))
        pl.when(c == 1)(lambda: run(ch1 // seg, _NS * ch0 + s * ch1))
        plsc.subcore_barrier()

        base = c * acc_rows + s * acc_rows_per_tile
        pltpu.sync_copy(acc_sh.at[pl.ds(s * acc_rows_per_tile, acc_rows_per_tile)],
                        out_hbm.at[pl.ds(base, acc_rows_per_tile)])

    return agg(h, src2d, dst2d)


def _emb_body(h_ref, w_ref, b_ref, o_ref):
    o_ref[...] = jnp.dot(h_ref[...], w_ref[...],
                         preferred_element_type=jnp.float32) + b_ref[...]


def _tc_emb(h, w, b):
    return pl.pallas_call(
        _emb_body,
        out_shape=jax.ShapeDtypeStruct((h.shape[0], w.shape[1]), jnp.float32),
    )(h, w, b.reshape(1, -1))


def _bn(x, gb_ref):
    g = gb_ref[0:1, :]
    b = gb_ref[1:2, :]
    m = jnp.mean(x, axis=0, keepdims=True)
    d = x - m
    v = jnp.mean(d * d, axis=0, keepdims=True)
    return g * (d * lax.rsqrt(v + 1e-5)) + b


def _layer_body(h_ref, p_ref, eps_ref, w1_ref, b1_ref,
                w2_ref, b2_ref, bn1_ref, bn2_ref, bn3_ref, o_ref):
    h = h_ref[...]
    n_rows = h_ref.shape[0]
    acc_rows = p_ref.shape[0] // 2
    p0 = p_ref[pl.ds(0, n_rows), :]
    p1 = p_ref[pl.ds(acc_rows, n_rows), :]
    hh = eps_ref[...] * h + (p0 + p1)
    y = jnp.dot(hh, w1_ref[...], preferred_element_type=jnp.float32) + b1_ref[...]
    y = jnp.maximum(_bn(y, bn1_ref), 0.0)
    y = jnp.dot(y, w2_ref[...], preferred_element_type=jnp.float32) + b2_ref[...]
    y = jnp.maximum(_bn(y, bn2_ref), 0.0)
    y = jnp.maximum(_bn(y, bn3_ref), 0.0)
    o_ref[...] = h + y


def _tc_layer(h, pfull, lp):
    (w1, b1), (w2, b2) = lp['mlp']
    epsp = (1.0 + lp['eps']).reshape(1, 1)
    bn1 = jnp.stack(lp['mlp_bn'])
    bn2 = jnp.stack(lp['apply_bn'])
    bn3 = jnp.stack(lp['layer_bn'])
    return pl.pallas_call(
        _layer_body,
        out_shape=jax.ShapeDtypeStruct(h.shape, jnp.float32),
    )(h, pfull, epsp, w1, b1.reshape(1, -1), w2, b2.reshape(1, -1),
      bn1, bn2, bn3)


def _readout_body(ids_ref, h0_ref, h1_ref, h2_ref, h3_ref, h4_ref,
                  wp_ref, bp_ref, wa_ref, ba_ref, wc_ref, bc_ref,
                  wo_ref, bo_ref, xc_ref, xo_ref, xco_ref):
    g_count = xc_ref.shape[0]
    n = ids_ref.shape[1]
    gi = lax.broadcasted_iota(jnp.int32, (g_count, n), 0)
    p_t = (gi == ids_ref[...]).astype(jnp.float32)

    score = jnp.zeros_like(xo_ref)
    hs = (h0_ref, h1_ref, h2_ref, h3_ref, h4_ref)
    pooled4 = None
    for i in range(5):
        pooled = jnp.dot(p_t, hs[i][...], preferred_element_type=jnp.float32)
        if i == 4:
            pooled4 = pooled
        score = score + jnp.dot(pooled, wp_ref[i],
                                preferred_element_type=jnp.float32) + bp_ref[i]

    hlast = h4_ref[...]
    d = jnp.dot(hlast, wa_ref[...],
                preferred_element_type=jnp.float32) + ba_ref[...]
    att0 = 1.0 / (1.0 + jnp.exp(-d))
    hc = jnp.dot(p_t, att0 * hlast, preferred_element_type=jnp.float32)
    ho = pooled4 - hc

    xc_ref[...] = jnp.dot(hc, wc_ref[...],
                          preferred_element_type=jnp.float32) + bc_ref[...]
    xo = jnp.dot(ho, wo_ref[...],
                 preferred_element_type=jnp.float32) + bo_ref[...] + score
    xo_ref[...] = xo
    hcr = jnp.concatenate([hc[g_count - 1:g_count], hc[:g_count - 1]], axis=0)
    xco_ref[...] = jnp.dot(ho + hcr, wo_ref[...],
                           preferred_element_type=jnp.float32) \
        + bo_ref[...] + score


def _tc_readout(ids_row, hidden, params, g_count, ncls):
    wp = jnp.stack([w for (w, _) in params['pred']])
    bp = jnp.stack([b for (_, b) in params['pred']])[:, None, :]
    wa, ba = params['att']
    wa_d = (wa[:, 0] - wa[:, 1]).reshape(-1, 1)
    ba_d = (ba[0] - ba[1]).reshape(1, 1)
    wc, bc = params['lc']
    wo, bo = params['lo']
    out_sh = jax.ShapeDtypeStruct((g_count, ncls), jnp.float32)
    return pl.pallas_call(
        _readout_body,
        out_shape=(out_sh, out_sh, out_sh),
    )(ids_row, *hidden, wp, bp, wa_d, ba_d, wc, bc.reshape(1, -1),
      wo, bo.reshape(1, -1))


def kernel(h, e, params, edge_index, node_graph_ids):
    n, _ = h.shape
    n_edges = edge_index.shape[1]
    g_count = 128
    ncls = params['lc'][0].shape[1]

    per_op = _NC * _NS * _LANES
    ch = -(-n_edges // (per_op * 8)) * 8
    e_pad = ch * per_op
    ch1 = ch
    ch0 = 2 * ch - ch1
    src = edge_index[0].astype(jnp.int32)
    dst = edge_index[1].astype(jnp.int32)
    pad = e_pad - n_edges
    acc_rows = -(-(n + 1) // (_NS * 8)) * (_NS * 8)
    pad_src = jnp.arange(pad, dtype=jnp.int32) % jnp.int32(n)
    pad_dst = n + jnp.arange(pad, dtype=jnp.int32) % jnp.int32(acc_rows - n)
    src2d = jnp.concatenate([src, pad_src]).reshape(-1, _LANES)
    dst2d = jnp.concatenate([dst, pad_dst]).reshape(-1, _LANES)

    w_emb, b_emb = params['emb']
    hcur = _tc_emb(h, w_emb, b_emb)
    hidden = [hcur]
    for lp in params['gin']:
        partials = _sc_edge_agg(hcur, src2d, dst2d,
                                n=n, acc_rows=acc_rows, ch0=ch0, ch1=ch1)
        hcur = _tc_layer(hcur, partials, lp)
        hidden.append(hcur)

    ids_row = node_graph_ids.astype(jnp.int32).reshape(1, n)
    return _tc_readout(ids_row, hidden, params, g_count, ncls)


# speedup vs baseline: 3.3735x; 1.0388x over previous
import functools

import jax
import jax.numpy as jnp
from jax import lax
from jax.experimental import pallas as pl
from jax.experimental.pallas import tpu as pltpu
from jax.experimental.pallas import tpu_sc as plsc

_NC = 2
_NS = 16
_LANES = 128


@functools.partial(jax.jit, static_argnames=("n", "acc_rows", "ch0", "ch1"))
def _sc_edge_agg(h, src2d, dst2d, *, n, acc_rows, ch0, ch1):
    hid = h.shape[1]
    acc_rows_per_tile = acc_rows // _NS
    mesh = plsc.VectorSubcoreMesh(
        core_axis_name="c", subcore_axis_name="s",
        num_cores=_NC, num_subcores=_NS)

    seg = 16
    assert ch0 % seg == 0 and ch1 % seg == 0

    @functools.partial(
        pl.kernel,
        out_type=jax.ShapeDtypeStruct((_NC * acc_rows, hid), jnp.float32),
        mesh=mesh,
        scratch_types=[
            pltpu.VMEM((seg, _LANES), jnp.int32),
            pltpu.VMEM((seg, _LANES), jnp.int32),
            pltpu.VMEM((seg, _LANES), jnp.int32),
            pltpu.VMEM((seg, _LANES), jnp.int32),
            pltpu.VMEM((_LANES, hid), jnp.float32),
            pltpu.VMEM((_LANES, hid), jnp.float32),
            pltpu.VMEM_SHARED((acc_rows, hid), jnp.float32),
            pltpu.SemaphoreType.DMA,
            pltpu.SemaphoreType.DMA,
            pltpu.SemaphoreType.DMA,
        ],
    )
    def agg(h_hbm, src_hbm, dst_hbm, out_hbm,
            src0, src1, dst0, dst1, buf0, buf1, acc_sh, g0, g1, isem):
        srcb = (src0, src1)
        dstb = (dst0, dst1)
        bufs = (buf0, buf1)
        gsems = (g0, g1)
        c = lax.axis_index("c")
        s = lax.axis_index("s")

        zv = jnp.zeros((16,), jnp.float32)

        def zrow(r, carry):
            for k in range(hid // 16):
                buf0[r, pl.ds(k * 16, 16)] = zv
            return carry

        lax.fori_loop(0, _LANES, zrow, 0)
        zbase = s * acc_rows_per_tile
        for k in range(acc_rows_per_tile // _LANES):
            pltpu.sync_copy(buf0,
                            acc_sh.at[pl.ds(zbase + k * _LANES, _LANES)])
        rem = acc_rows_per_tile % _LANES
        if rem:
            full = acc_rows_per_tile - rem
            pltpu.sync_copy(buf0.at[pl.ds(0, rem)],
                            acc_sh.at[pl.ds(zbase + full, rem)])
        plsc.subcore_barrier()

        def step(src_v, dst_v, j, b, pf_src, pf_j):
            pltpu.make_async_copy(h_hbm.at[src_v.at[j]], bufs[b],
                                  gsems[b]).wait()
            pltpu.sync_copy(bufs[b], acc_sh.at[dst_v.at[j]], add=True)
            if pf_src is not None:
                pltpu.async_copy(h_hbm.at[pf_src.at[pf_j]], bufs[b],
                                 gsems[b])

        def run(nseg, ebase):
            pltpu.sync_copy(src_hbm.at[pl.ds(ebase, seg)], src0)
            pltpu.sync_copy(dst_hbm.at[pl.ds(ebase, seg)], dst0)
            pltpu.async_copy(h_hbm.at[src0.at[0]], bufs[0], gsems[0])
            pltpu.async_copy(h_hbm.at[src0.at[1]], bufs[1], gsems[1])
            for sg in range(nseg):
                p = sg % 2
                src_v, dst_v = srcb[p], dstb[p]
                if sg + 1 < nseg:
                    off = ebase + (sg + 1) * seg
                    pltpu.async_copy(src_hbm.at[pl.ds(off, seg)],
                                     srcb[1 - p], isem)
                    pltpu.async_copy(dst_hbm.at[pl.ds(off, seg)],
                                     dstb[1 - p], isem)

                def body(i, carry):
                    step(src_v, dst_v, 2 * i, 0, src_v, 2 * i + 2)
                    step(src_v, dst_v, 2 * i + 1, 1, src_v, 2 * i + 3)
                    return carry

                lax.fori_loop(0, seg // 2 - 1, body, 0)
                if sg + 1 < nseg:
                    pltpu.make_async_copy(
                        src_hbm.at[pl.ds(ebase, seg)], srcb[1 - p],
                        isem).wait()
                    pltpu.make_async_copy(
                        dst_hbm.at[pl.ds(ebase, seg)], dstb[1 - p],
                        isem).wait()
                    step(src_v, dst_v, seg - 2, 0, srcb[1 - p], 0)
                    step(src_v, dst_v, seg - 1, 1, srcb[1 - p], 1)
                else:
                    step(src_v, dst_v, seg - 2, 0, None, 0)
                    step(src_v, dst_v, seg - 1, 1, None, 0)

        pl.when(c == 0)(lambda: run(ch0 // seg, s * ch0))
        pl.when(c == 1)(lambda: run(ch1 // seg, _NS * ch0 + s * ch1))
        plsc.subcore_barrier()

        base = c * acc_rows + s * acc_rows_per_tile
        pltpu.sync_copy(acc_sh.at[pl.ds(s * acc_rows_per_tile, acc_rows_per_tile)],
                        out_hbm.at[pl.ds(base, acc_rows_per_tile)])

    return agg(h, src2d, dst2d)


def _emb_body(h_ref, w_ref, b_ref, o_ref):
    o_ref[...] = jnp.dot(h_ref[...], w_ref[...],
                         preferred_element_type=jnp.float32) + b_ref[...]


def _tc_emb(h, w, b):
    return pl.pallas_call(
        _emb_body,
        out_shape=jax.ShapeDtypeStruct((h.shape[0], w.shape[1]), jnp.float32),
    )(h, w, b.reshape(1, -1))


def _bn(x, gb_ref):
    g = gb_ref[0:1, :]
    b = gb_ref[1:2, :]
    m = jnp.mean(x, axis=0, keepdims=True)
    d = x - m
    v = jnp.mean(d * d, axis=0, keepdims=True)
    return g * (d * lax.rsqrt(v + 1e-5)) + b


def _layer_body(h_ref, p_ref, eps_ref, w1_ref, b1_ref,
                w2_ref, b2_ref, bn1_ref, bn2_ref, bn3_ref, o_ref):
    h = h_ref[...]
    n_rows = h_ref.shape[0]
    acc_rows = p_ref.shape[0] // 2
    p0 = p_ref[pl.ds(0, n_rows), :]
    p1 = p_ref[pl.ds(acc_rows, n_rows), :]
    hh = eps_ref[...] * h + (p0 + p1)
    y = jnp.dot(hh, w1_ref[...], preferred_element_type=jnp.float32) + b1_ref[...]
    y = jnp.maximum(_bn(y, bn1_ref), 0.0)
    y = jnp.dot(y, w2_ref[...], preferred_element_type=jnp.float32) + b2_ref[...]
    y = jnp.maximum(_bn(y, bn2_ref), 0.0)
    y = jnp.maximum(_bn(y, bn3_ref), 0.0)
    o_ref[...] = h + y


def _tc_layer(h, pfull, lp):
    (w1, b1), (w2, b2) = lp['mlp']
    epsp = (1.0 + lp['eps']).reshape(1, 1)
    bn1 = jnp.stack(lp['mlp_bn'])
    bn2 = jnp.stack(lp['apply_bn'])
    bn3 = jnp.stack(lp['layer_bn'])
    return pl.pallas_call(
        _layer_body,
        out_shape=jax.ShapeDtypeStruct(h.shape, jnp.float32),
    )(h, pfull, epsp, w1, b1.reshape(1, -1), w2, b2.reshape(1, -1),
      bn1, bn2, bn3)


def _readout_body(ids_ref, h0_ref, h1_ref, h2_ref, h3_ref, h4_ref,
                  wp_ref, bp_ref, wa_ref, ba_ref, wc_ref, bc_ref,
                  wo_ref, bo_ref, xc_ref, xo_ref, xco_ref):
    g_count = xc_ref.shape[0]
    n = ids_ref.shape[1]
    gi = lax.broadcasted_iota(jnp.int32, (g_count, n), 0)
    p_t = (gi == ids_ref[...]).astype(jnp.float32)

    score = jnp.zeros_like(xo_ref)
    hs = (h0_ref, h1_ref, h2_ref, h3_ref, h4_ref)
    pooled4 = None
    for i in range(5):
        pooled = jnp.dot(p_t, hs[i][...], preferred_element_type=jnp.float32)
        if i == 4:
            pooled4 = pooled
        score = score + jnp.dot(pooled, wp_ref[i],
                                preferred_element_type=jnp.float32) + bp_ref[i]

    hlast = h4_ref[...]
    d = jnp.dot(hlast, wa_ref[...],
                preferred_element_type=jnp.float32) + ba_ref[...]
    att0 = 1.0 / (1.0 + jnp.exp(-d))
    hc = jnp.dot(p_t, att0 * hlast, preferred_element_type=jnp.float32)
    ho = pooled4 - hc

    xc_ref[...] = jnp.dot(hc, wc_ref[...],
                          preferred_element_type=jnp.float32) + bc_ref[...]
    xo = jnp.dot(ho, wo_ref[...],
                 preferred_element_type=jnp.float32) + bo_ref[...] + score
    xo_ref[...] = xo
    hcr = jnp.concatenate([hc[g_count - 1:g_count], hc[:g_count - 1]], axis=0)
    xco_ref[...] = jnp.dot(ho + hcr, wo_ref[...],
                           preferred_element_type=jnp.float32) \
        + bo_ref[...] + score


def _tc_readout(ids_row, hidden, params, g_count, ncls):
    wp = jnp.stack([w for (w, _) in params['pred']])
    bp = jnp.stack([b for (_, b) in params['pred']])[:, None, :]
    wa, ba = params['att']
    wa_d = (wa[:, 0] - wa[:, 1]).reshape(-1, 1)
    ba_d = (ba[0] - ba[1]).reshape(1, 1)
    wc, bc = params['lc']
    wo, bo = params['lo']
    out_sh = jax.ShapeDtypeStruct((g_count, ncls), jnp.float32)
    return pl.pallas_call(
        _readout_body,
        out_shape=(out_sh, out_sh, out_sh),
    )(ids_row, *hidden, wp, bp, wa_d, ba_d, wc, bc.reshape(1, -1),
      wo, bo.reshape(1, -1))


def kernel(h, e, params, edge_index, node_graph_ids):
    n, _ = h.shape
    n_edges = edge_index.shape[1]
    g_count = 128
    ncls = params['lc'][0].shape[1]

    per_op = _NC * _NS * _LANES
    ch = -(-n_edges // (per_op * 8)) * 8
    e_pad = ch * per_op
    ch1 = ch
    ch0 = 2 * ch - ch1
    src = edge_index[0].astype(jnp.int32)
    dst = edge_index[1].astype(jnp.int32)
    pad = e_pad - n_edges
    acc_rows = -(-(n + 1) // (_NS * 8)) * (_NS * 8)
    pad_src = jnp.arange(pad, dtype=jnp.int32) % jnp.int32(n)
    pad_dst = n + jnp.arange(pad, dtype=jnp.int32) % jnp.int32(acc_rows - n)
    src2d = jnp.concatenate([src, pad_src]).reshape(-1, _LANES)
    dst2d = jnp.concatenate([dst, pad_dst]).reshape(-1, _LANES)

    w_emb, b_emb = params['emb']
    hcur = _tc_emb(h, w_emb, b_emb)
    hidden = [hcur]
    for lp in params['gin']:
        partials = _sc_edge_agg(hcur, src2d, dst2d,
                                n=n, acc_rows=acc_rows, ch0=ch0, ch1=ch1)
        hcur = _tc_layer(hcur, partials, lp)
        hidden.append(hcur)

    ids_row = node_graph_ids.astype(jnp.int32).reshape(1, n)
    return _tc_readout(ids_row, hidden, params, g_count, ncls)
